# Initial kernel scaffold; baseline (speedup 1.0000x reference)
import jax
import jax.numpy as jnp
from jax.experimental import pallas as pl


def kernel(x_nchw, w_oihw, gamma, beta):
    raise NotImplementedError("write your pallas kernel here")


# speedup vs baseline: 1.5173x; 1.5173x over previous
import numpy as np

import jax
import jax.numpy as jnp
from jax import lax
from jax.experimental import pallas as pl
from jax.experimental.pallas import tpu as pltpu

_SMEM = pl.BlockSpec(memory_space=pltpu.MemorySpace.SMEM)

_PAIR_IDX = np.zeros((9, 9), dtype=np.int32)
_k = 0
for _a in range(9):
    for _b in range(_a, 9):
        _PAIR_IDX[_a, _b] = _PAIR_IDX[_b, _a] = _k
        _k += 1


def _tap_list(xph_ref, Ho, Wo):
    taps = []
    for dh in range(3):
        for dw in range(3):
            p = (dh % 2) * 2 + (dw % 2)
            r0, c0 = dh // 2, dw // 2
            taps.append(xph_ref[:, p, r0:r0 + Ho, c0:c0 + Wo])
    return taps


def _pool_plane(taps, Ho, Wo, h_odd, w_odd):
    row = lax.broadcasted_iota(jnp.int32, (Ho, Wo), 0)
    col = lax.broadcasted_iota(jnp.int32, (Ho, Wo), 1)
    neg = jnp.float32(-jnp.inf)

    def row_bad(dh):
        m = None
        if dh == 0:
            m = row == 0
        if dh == 2 and h_odd:
            m = row == Ho - 1 if m is None else m | (row == Ho - 1)
        return m

    def col_bad(dw):
        m = None
        if dw == 0:
            m = col == 0
        if dw == 2 and w_odd:
            m = col == Wo - 1 if m is None else m | (col == Wo - 1)
        return m

    groups = {}
    for dh in range(3):
        for dw in range(3):
            rm, cm = row_bad(dh), col_bad(dw)
            key = (rm is not None, cm is not None,
                   dh if rm is not None else -1,
                   dw if cm is not None else -1)
            groups.setdefault(key, []).append(taps[3 * dh + dw])

    out = None
    for (has_r, has_c, dh, dw), ts in groups.items():
        g = ts[0]
        for t in ts[1:]:
            g = jnp.maximum(g, t)
        mask = None
        if has_r:
            mask = row_bad(dh)
        if has_c:
            cm = col_bad(dw)
            mask = cm if mask is None else mask | cm
        if mask is not None:
            g = jnp.where(mask[None, :, :], neg, g)
        out = g if out is None else jnp.maximum(out, g)
    return out


def _make_stats_kernel(Ho, Wo, h_odd, w_odd):
    def stats_kernel(xph_ref, acc_ref):
        @pl.when(pl.program_id(0) == 0)
        def _init():
            acc_ref[...] = jnp.zeros_like(acc_ref)

        taps = _tap_list(xph_ref, Ho, Wo)
        rows = [jnp.sum(t, axis=(0, 1)) for t in taps]
        for a in range(9):
            for b in range(a, 9):
                rows.append(jnp.sum(taps[a] * taps[b], axis=(0, 1)))
        pool = _pool_plane(taps, Ho, Wo, h_odd, w_odd)
        rows.append(jnp.sum(pool, axis=(0, 1)))
        rows.append(jnp.sum(pool * pool, axis=(0, 1)))
        acc_ref[...] += jnp.stack(rows, axis=0)

    return stats_kernel


def _make_apply_kernel(Ho, Wo, Cm, h_odd, w_odd):
    def apply_kernel(w_ref, aff_ref, xph_ref, out_ref):
        taps = _tap_list(xph_ref, Ho, Wo)
        for c in range(Cm):
            acc = taps[0] * w_ref[c, 0]
            for k in range(1, 9):
                acc = acc + taps[k] * w_ref[c, k]
            out_ref[:, c, :, :] = jnp.maximum(acc + aff_ref[1, c], 0.0)
        pool = _pool_plane(taps, Ho, Wo, h_odd, w_odd)
        out_ref[:, Cm, :, :] = jnp.maximum(
            pool * aff_ref[0, Cm] + aff_ref[1, Cm], 0.0)

    return apply_kernel


def kernel(x_nchw, w_oihw, gamma, beta, eps=1e-5):
    N, Cin, H, W = x_nchw.shape
    Cm = w_oihw.shape[0]
    Cout = Cm + 1
    Ho = (H - 1) // 2 + 1
    Wo = (W - 1) // 2 + 1
    Hp = 2 * (Ho + 1)
    Wp = 2 * (Wo + 1)
    h_odd = 2 * (Ho - 1) + 2 >= H + 1
    w_odd = 2 * (Wo - 1) + 2 >= W + 1

    x2d = x_nchw.reshape(N, H, W)
    xpad = jnp.pad(x2d, ((0, 0), (1, Hp - H - 1), (1, Wp - W - 1)))
    xph = (xpad.reshape(N, Ho + 1, 2, Wo + 1, 2)
               .transpose(0, 2, 4, 1, 3)
               .reshape(N, 4, Ho + 1, Wo + 1))

    nb = int(min(N, 8))
    n_pad = (-N) % nb
    if n_pad:
        xph = jnp.pad(xph, ((0, n_pad), (0, 0), (0, 0), (0, 0)))
    steps = (N + n_pad) // nb

    xph_spec = pl.BlockSpec((nb, 4, Ho + 1, Wo + 1), lambda n: (n, 0, 0, 0))

    acc = pl.pallas_call(
        _make_stats_kernel(Ho, Wo, h_odd, w_odd),
        out_shape=jax.ShapeDtypeStruct((56, Wo), jnp.float32),
        grid=(steps,),
        in_specs=[xph_spec],
        out_specs=pl.BlockSpec((56, Wo), lambda n: (0, 0)),
        compiler_params=pltpu.CompilerParams(
            dimension_semantics=("arbitrary",),
            vmem_limit_bytes=100 << 20),
    )(xph)

    v = jnp.sum(acc, axis=1)
    tap_s = v[:9]
    gram = v[9:54][_PAIR_IDX.reshape(-1)].reshape(9, 9)
    w_flat = w_oihw.reshape(Cm, 9).astype(jnp.float32)
    cnt = jnp.float32(N * Ho * Wo)

    s_all = jnp.concatenate([w_flat @ tap_s, v[54:55]])
    ss_all = jnp.concatenate(
        [jnp.einsum("ck,kl,cl->c", w_flat, gram, w_flat), v[55:56]])
    mean = s_all / cnt
    var = jnp.maximum(ss_all / cnt - mean * mean, 0.0)
    scale = gamma.astype(jnp.float32) * lax.rsqrt(var + eps)
    shift = beta.astype(jnp.float32) - mean * scale
    w_scaled = w_flat * scale[:Cm, None]
    aff = jnp.stack([scale, shift], axis=0)

    out = pl.pallas_call(
        _make_apply_kernel(Ho, Wo, Cm, h_odd, w_odd),
        out_shape=jax.ShapeDtypeStruct((N + n_pad, Cout, Ho, Wo), jnp.float32),
        grid=(steps,),
        in_specs=[_SMEM, _SMEM, xph_spec],
        out_specs=pl.BlockSpec((nb, Cout, Ho, Wo), lambda n: (n, 0, 0, 0)),
        compiler_params=pltpu.CompilerParams(
            dimension_semantics=("parallel",),
            vmem_limit_bytes=100 << 20),
    )(w_scaled, aff, xph)

    return out[:N] if n_pad else out


# speedup vs baseline: 1.9506x; 1.2855x over previous
import numpy as np

import jax
import jax.numpy as jnp
from jax import lax
from jax.experimental import pallas as pl
from jax.experimental.pallas import tpu as pltpu

_SMEM = pl.BlockSpec(memory_space=pltpu.MemorySpace.SMEM)

_PAIR_IDX = np.zeros((9, 9), dtype=np.int32)
_k = 0
for _a in range(9):
    for _b in range(_a, 9):
        _PAIR_IDX[_a, _b] = _PAIR_IDX[_b, _a] = _k
        _k += 1


def _sel_matrix(W, Wo, osz):
    S = np.zeros((2 * W, 6 * osz), dtype=np.float32)
    for j in range(Wo):
        for half, base in ((0, 0), (1, 3 * osz)):
            for dw in range(3):
                src = 2 * j + dw - 1
                if 0 <= src < W:
                    S[half * W + src, base + dw * osz + j] = 1.0
    return S


def _build_taps(x_ref, s_ref, nb, H, W, Ho, Wo, osz):
    f32 = jnp.float32
    y = jnp.dot(x_ref[...].reshape(nb * Ho, 2 * W), s_ref[...],
                preferred_element_type=f32).reshape(nb, Ho, 6 * osz)

    def shift1(t):
        return jnp.concatenate(
            [jnp.zeros((nb, 1, Wo), f32), t[:, :Ho - 1, :]], axis=1)

    taps = [None] * 9
    for dw in range(3):
        t1 = y[:, :, dw * osz: dw * osz + Wo]
        t2 = y[:, :, (3 + dw) * osz: (3 + dw) * osz + Wo]
        taps[3 * 1 + dw] = t1
        taps[3 * 2 + dw] = t2
        taps[3 * 0 + dw] = shift1(t2)
    return taps


def _pool_plane(taps, Ho, Wo):
    row = lax.broadcasted_iota(jnp.int32, (Ho, Wo), 0)
    col = lax.broadcasted_iota(jnp.int32, (Ho, Wo), 1)
    top = (row == 0)[None]
    left = (col == 0)[None]
    neg = jnp.float32(-jnp.inf)

    def gmax(ts):
        g = ts[0]
        for t in ts[1:]:
            g = jnp.maximum(g, t)
        return g

    inner = gmax([taps[3 * dh + dw]
                  for dh in (1, 2) for dw in (1, 2)])
    toprow = jnp.where(top, neg, gmax([taps[1], taps[2]]))
    leftcol = jnp.where(left, neg, gmax([taps[3], taps[6]]))
    corner = jnp.where(top | left, neg, taps[0])
    return jnp.maximum(jnp.maximum(inner, toprow),
                       jnp.maximum(leftcol, corner))


def _make_stats_kernel(nb, H, W, Ho, Wo, osz):
    def stats_kernel(s_ref, x_ref, acc_ref):
        @pl.when(pl.program_id(0) == 0)
        def _init():
            acc_ref[...] = jnp.zeros_like(acc_ref)

        taps = _build_taps(x_ref, s_ref, nb, H, W, Ho, Wo, osz)
        rows = [jnp.sum(t, axis=(0, 1)) for t in taps]
        for a in range(9):
            for b in range(a, 9):
                rows.append(jnp.sum(taps[a] * taps[b], axis=(0, 1)))
        pool = _pool_plane(taps, Ho, Wo)
        rows.append(jnp.sum(pool, axis=(0, 1)))
        rows.append(jnp.sum(pool * pool, axis=(0, 1)))
        acc_ref[...] += jnp.stack(rows, axis=0)

    return stats_kernel


def _make_apply_kernel(nb, H, W, Ho, Wo, osz, Cm):
    def apply_kernel(w_ref, aff_ref, s_ref, x_ref, out_ref):
        taps = _build_taps(x_ref, s_ref, nb, H, W, Ho, Wo, osz)
        for c in range(Cm):
            acc = taps[0] * w_ref[c, 0]
            for k in range(1, 9):
                acc = acc + taps[k] * w_ref[c, k]
            out_ref[:, c, :, :] = jnp.maximum(acc + aff_ref[1, c], 0.0)
        pool = _pool_plane(taps, Ho, Wo)
        out_ref[:, Cm, :, :] = jnp.maximum(
            pool * aff_ref[0, Cm] + aff_ref[1, Cm], 0.0)

    return apply_kernel


def kernel(x_nchw, w_oihw, gamma, beta, eps=1e-5):
    N, Cin, H, W = x_nchw.shape
    assert Cin == 1 and H % 2 == 0 and W % 2 == 0
    Cm = w_oihw.shape[0]
    Cout = Cm + 1
    Ho, Wo = H // 2, W // 2
    osz = 128
    assert Wo <= osz

    nb = int(min(N, 8))
    n_pad = (-N) % nb
    if n_pad:
        x_nchw = jnp.pad(x_nchw, ((0, n_pad), (0, 0), (0, 0), (0, 0)))
    steps = (N + n_pad) // nb

    sel = jnp.asarray(_sel_matrix(W, Wo, osz))
    x3 = x_nchw.reshape(N + n_pad, Ho, 2 * W)
    x_spec = pl.BlockSpec((nb, Ho, 2 * W), lambda n: (n, 0, 0))
    s_spec = pl.BlockSpec((2 * W, 6 * osz), lambda n: (0, 0))

    acc = pl.pallas_call(
        _make_stats_kernel(nb, H, W, Ho, Wo, osz),
        out_shape=jax.ShapeDtypeStruct((56, Wo), jnp.float32),
        grid=(steps,),
        in_specs=[s_spec, x_spec],
        out_specs=pl.BlockSpec((56, Wo), lambda n: (0, 0)),
        compiler_params=pltpu.CompilerParams(
            dimension_semantics=("arbitrary",),
            vmem_limit_bytes=100 << 20),
    )(sel, x3)

    v = jnp.sum(acc, axis=1)
    tap_s = v[:9]
    gram = v[9:54][_PAIR_IDX.reshape(-1)].reshape(9, 9)
    w_flat = w_oihw.reshape(Cm, 9).astype(jnp.float32)
    cnt = jnp.float32(N * Ho * Wo)

    s_all = jnp.concatenate([w_flat @ tap_s, v[54:55]])
    ss_all = jnp.concatenate(
        [jnp.einsum("ck,kl,cl->c", w_flat, gram, w_flat), v[55:56]])
    mean = s_all / cnt
    var = jnp.maximum(ss_all / cnt - mean * mean, 0.0)
    scale = gamma.astype(jnp.float32) * lax.rsqrt(var + eps)
    shift = beta.astype(jnp.float32) - mean * scale
    w_scaled = w_flat * scale[:Cm, None]
    aff = jnp.stack([scale, shift], axis=0)

    out = pl.pallas_call(
        _make_apply_kernel(nb, H, W, Ho, Wo, osz, Cm),
        out_shape=jax.ShapeDtypeStruct((N + n_pad, Cout, Ho, Wo), jnp.float32),
        grid=(steps,),
        in_specs=[_SMEM, _SMEM, s_spec, x_spec],
        out_specs=pl.BlockSpec((nb, Cout, Ho, Wo), lambda n: (n, 0, 0, 0)),
        compiler_params=pltpu.CompilerParams(
            dimension_semantics=("arbitrary",),
            vmem_limit_bytes=100 << 20),
    )(w_scaled, aff, sel, x3)

    return out[:N] if n_pad else out
